# Initial kernel scaffold; baseline (speedup 1.0000x reference)
import jax
import jax.numpy as jnp
from jax.experimental import pallas as pl


def kernel(text_ids, table, W1, b1, W2, b2):
    raise NotImplementedError("write your pallas kernel here")


# speedup vs baseline: 7.5215x; 7.5215x over previous
import functools

import jax
import jax.numpy as jnp
from jax import lax
from jax.experimental import pallas as pl
from jax.experimental.pallas import tpu as pltpu
from jax.experimental.pallas import tpu_sc as plsc

NC = 2
NS = 16
NW = NC * NS
LANE = 16

def _seq_chunks(seq):
    chunks = [128] * (seq // 128)
    if seq % 128:
        chunks.append(seq % 128)
    return chunks


def _sc_pool_body(seq, embed, bpw, ids_hbm, table_hbm, out_hbm,
                  idx_v, buf_v, out_v, sem):
    nvec = embed // LANE
    wid = lax.axis_index("s") * NC + lax.axis_index("c")
    base = wid * bpw
    pltpu.sync_copy(ids_hbm.at[pl.ds(base * seq, bpw * seq)], idx_v)

    def row(b, _):
        off = b * seq
        descs = []
        dst = 0
        for c in _seq_chunks(seq):
            descs.append(pltpu.async_copy(
                table_hbm.at[idx_v.at[pl.ds(off + dst, c)]],
                buf_v.at[pl.ds(dst, c)], sem))
            dst += c
        for d in descs:
            d.wait()

        def acc_body(s, acc):
            return tuple(acc[j] + buf_v[s, pl.ds(j * LANE, LANE)]
                         for j in range(nvec))

        acc = lax.fori_loop(
            0, seq, acc_body,
            tuple(jnp.zeros((LANE,), jnp.float32) for _ in range(nvec)))
        for j in range(nvec):
            out_v[b, pl.ds(j * LANE, LANE)] = acc[j]
        return 0

    lax.fori_loop(0, bpw, row, 0)
    pltpu.sync_copy(out_v, out_hbm.at[pl.ds(base, bpw)])


def _make_sc_pool(batch, seq, vocab, embed, interpret=False):
    bpw = batch // NW
    mesh = plsc.VectorSubcoreMesh(
        core_axis_name="c", subcore_axis_name="s",
        num_cores=NC, num_subcores=NS)
    return functools.partial(
        pl.kernel,
        out_type=jax.ShapeDtypeStruct((batch, embed), jnp.float32),
        mesh=mesh,
        scratch_types=[
            pltpu.VMEM((bpw * seq,), jnp.int32),
            pltpu.VMEM((seq, embed), jnp.float32),
            pltpu.VMEM((bpw, embed), jnp.float32),
            pltpu.SemaphoreType.DMA,
        ],
        interpret=interpret,
    )(functools.partial(_sc_pool_body, seq, embed, bpw))


def _mlp_body(x_ref, w1_ref, b1_ref, w2t_ref, b2_ref, o_ref):
    x = x_ref[...]
    h = jnp.dot(x, w1_ref[...], preferred_element_type=jnp.float32)
    h = jnp.maximum(h + b1_ref[...], 0.0)
    logit = jnp.sum(h * w2t_ref[...], axis=1, keepdims=True) + b2_ref[0, 0]
    o_ref[...] = 1.0 / (1.0 + jnp.exp(-logit))


def _mlp(encoded, W1, b1, W2, b2, interpret=False):
    batch, embed = encoded.shape
    hidden = W1.shape[1]
    mb = min(512, batch)
    return pl.pallas_call(
        _mlp_body,
        grid=(batch // mb,),
        in_specs=[
            pl.BlockSpec((mb, embed), lambda i: (i, 0)),
            pl.BlockSpec((embed, hidden), lambda i: (0, 0)),
            pl.BlockSpec((1, hidden), lambda i: (0, 0)),
            pl.BlockSpec((1, hidden), lambda i: (0, 0)),
            pl.BlockSpec((1, 1), lambda i: (0, 0)),
        ],
        out_specs=pl.BlockSpec((mb, 1), lambda i: (i, 0)),
        out_shape=jax.ShapeDtypeStruct((batch, 1), jnp.float32),
        interpret=interpret,
    )(encoded, W1, b1[None, :], W2.T, b2[None, :])


def kernel(text_ids, table, W1, b1, W2, b2):
    batch, seq = text_ids.shape
    vocab, embed = table.shape
    ids_flat = text_ids.reshape(-1).astype(jnp.int32)
    encoded = _make_sc_pool(batch, seq, vocab, embed)(ids_flat, table)
    return _mlp(encoded, W1, b1, W2, b2)


# speedup vs baseline: 12.9736x; 1.7249x over previous
import functools

import jax
import jax.numpy as jnp
from jax import lax
from jax.experimental import pallas as pl
from jax.experimental.pallas import tpu as pltpu
from jax.experimental.pallas import tpu_sc as plsc

NC = 2
NS = 16
NW = NC * NS
LANE = 16

def _seq_chunks(seq):
    chunks = [128] * (seq // 128)
    if seq % 128:
        chunks.append(seq % 128)
    return chunks


ACC_UNROLL = 8


def _sc_pool_body(seq, embed, bpw, ids_hbm, table_hbm, out_hbm,
                  idx_v, buf0, buf1, out_v, sem0, sem1):
    nvec = embed // LANE
    chunks = _seq_chunks(seq)
    bufs, sems = (buf0, buf1), (sem0, sem1)
    wid = lax.axis_index("s") * NC + lax.axis_index("c")
    base = wid * bpw
    pltpu.sync_copy(ids_hbm.at[pl.ds(base * seq, bpw * seq)], idx_v)

    def fire(b, p):
        off = b * seq
        dst = 0
        for c in chunks:
            pltpu.async_copy(table_hbm.at[idx_v.at[pl.ds(off + dst, c)]],
                             bufs[p].at[pl.ds(dst, c)], sems[p])
            dst += c

    def drain(b, p):
        off = b * seq
        dst = 0
        for c in chunks:
            pltpu.make_async_copy(table_hbm.at[idx_v.at[pl.ds(off + dst, c)]],
                                  bufs[p].at[pl.ds(dst, c)], sems[p]).wait()
            dst += c

    fire(0, 0)
    fire(1, 1)

    zeros = tuple(jnp.zeros((LANE,), jnp.float32) for _ in range(nvec))
    n_groups, rem = divmod(seq, ACC_UNROLL)

    def body(i, _):
        for p in range(2):
            b = 2 * i + p
            drain(b, p)

            def acc_body(g, acc, p=p):
                for u in range(ACC_UNROLL):
                    s = g * ACC_UNROLL + u
                    acc = tuple(acc[j] + bufs[p][s, pl.ds(j * LANE, LANE)]
                                for j in range(nvec))
                return acc

            acc = lax.fori_loop(0, n_groups, acc_body, zeros)
            for s in range(seq - rem, seq):
                acc = tuple(acc[j] + bufs[p][s, pl.ds(j * LANE, LANE)]
                            for j in range(nvec))
            for j in range(nvec):
                out_v[b, pl.ds(j * LANE, LANE)] = acc[j]

            nb = b + 2

            @pl.when(nb < bpw)
            def _(nb=nb, p=p):
                fire(nb, p)
        return 0

    lax.fori_loop(0, bpw // 2, body, 0)
    pltpu.sync_copy(out_v, out_hbm.at[pl.ds(base, bpw)])


def _make_sc_pool(batch, seq, vocab, embed, interpret=False):
    bpw = batch // NW
    mesh = plsc.VectorSubcoreMesh(
        core_axis_name="c", subcore_axis_name="s",
        num_cores=NC, num_subcores=NS)
    return functools.partial(
        pl.kernel,
        out_type=jax.ShapeDtypeStruct((batch, embed), jnp.float32),
        mesh=mesh,
        scratch_types=[
            pltpu.VMEM((bpw * seq,), jnp.int32),
            pltpu.VMEM((seq, embed), jnp.float32),
            pltpu.VMEM((seq, embed), jnp.float32),
            pltpu.VMEM((bpw, embed), jnp.float32),
            pltpu.SemaphoreType.DMA,
            pltpu.SemaphoreType.DMA,
        ],
        interpret=interpret,
    )(functools.partial(_sc_pool_body, seq, embed, bpw))


def _mlp_body(x_ref, w1_ref, b1_ref, w2t_ref, b2_ref, o_ref):
    x = x_ref[...]
    h = jnp.dot(x, w1_ref[...], preferred_element_type=jnp.float32)
    h = jnp.maximum(h + b1_ref[...], 0.0)
    logit = jnp.sum(h * w2t_ref[...], axis=1, keepdims=True) + b2_ref[0, 0]
    o_ref[...] = 1.0 / (1.0 + jnp.exp(-logit))


def _mlp(encoded, W1, b1, W2, b2, interpret=False):
    batch, embed = encoded.shape
    hidden = W1.shape[1]
    mb = min(512, batch)
    return pl.pallas_call(
        _mlp_body,
        grid=(batch // mb,),
        in_specs=[
            pl.BlockSpec((mb, embed), lambda i: (i, 0)),
            pl.BlockSpec((embed, hidden), lambda i: (0, 0)),
            pl.BlockSpec((1, hidden), lambda i: (0, 0)),
            pl.BlockSpec((1, hidden), lambda i: (0, 0)),
            pl.BlockSpec((1, 1), lambda i: (0, 0)),
        ],
        out_specs=pl.BlockSpec((mb, 1), lambda i: (i, 0)),
        out_shape=jax.ShapeDtypeStruct((batch, 1), jnp.float32),
        interpret=interpret,
    )(encoded, W1, b1[None, :], W2.T, b2[None, :])


def kernel(text_ids, table, W1, b1, W2, b2):
    batch, seq = text_ids.shape
    vocab, embed = table.shape
    ids_flat = text_ids.reshape(-1).astype(jnp.int32)
    encoded = _make_sc_pool(batch, seq, vocab, embed)(ids_flat, table)
    return _mlp(encoded, W1, b1, W2, b2)
